# baseline (speedup 1.0000x reference)
import functools

import jax
import jax.numpy as jnp
from jax import lax
from jax.experimental import pallas as pl
from jax.experimental.pallas import tpu as pltpu
from jax.experimental.pallas import tpu_sc as plsc

B = 4096
L = 200
EMB = 64
HID = 128

NC = 2
NS = 16
NW = NC * NS
BPW = B // NW
C0 = 104
C1 = L - C0

_mesh = plsc.VectorSubcoreMesh(core_axis_name="c", subcore_axis_name="s")


@functools.partial(
    pl.kernel,
    out_type=jax.ShapeDtypeStruct((B, EMB), jnp.float32),
    mesh=_mesh,
    compiler_params=pltpu.CompilerParams(use_tc_tiling_on_sc=False),
    scratch_types=[
        pltpu.VMEM((BPW * L,), jnp.int32),
        pltpu.VMEM((L, EMB), jnp.float32),
        pltpu.VMEM((BPW, EMB), jnp.float32),
        pltpu.SemaphoreType.DMA,
    ],
)
def _pool_sum(x_hbm, table_hbm, out_hbm, idx_v, rows_v, acc_v, sem):
    wid = lax.axis_index("s") * NC + lax.axis_index("c")
    base = wid * (BPW * L)
    pltpu.sync_copy(x_hbm.at[pl.ds(base, BPW * L)], idx_v)

    def row_body(i, _):
        off = i * L
        h1 = pltpu.async_copy(
            table_hbm.at[idx_v.at[pl.ds(off, C0)]], rows_v.at[pl.ds(0, C0)], sem
        )
        h2 = pltpu.async_copy(
            table_hbm.at[idx_v.at[pl.ds(off + C0, C1)]],
            rows_v.at[pl.ds(C0, C1)],
            sem,
        )
        h1.wait()
        h2.wait()

        def red(j, accs):
            return tuple(
                accs[k] + rows_v[j, pl.ds(k * 16, 16)] for k in range(EMB // 16)
            )

        zero = jnp.zeros((16,), jnp.float32)
        accs = lax.fori_loop(0, L, red, (zero,) * (EMB // 16))
        for k in range(EMB // 16):
            acc_v[i, pl.ds(k * 16, 16)] = accs[k]
        return 0

    lax.fori_loop(0, BPW, row_body, 0)
    pltpu.sync_copy(acc_v, out_hbm.at[pl.ds(wid * BPW, BPW)])


def _mlp_body(pooled_ref, w1_ref, b1_ref, w2_ref, b2_ref, out_ref):
    s = jnp.dot(pooled_ref[...], w1_ref[...], preferred_element_type=jnp.float32)
    h = jnp.maximum(s * (1.0 / L) + b1_ref[...], 0.0)
    o = jnp.dot(h, w2_ref[...], preferred_element_type=jnp.float32)
    out_ref[...] = o + b2_ref[...]


_mlp = pl.pallas_call(
    _mlp_body,
    out_shape=jax.ShapeDtypeStruct((B, 1), jnp.float32),
)


def kernel(x, table, W1, b1, W2, b2):
    x_flat = x.reshape(-1).astype(jnp.int32)
    sums = _pool_sum(x_flat, table)
    out = _mlp(sums, W1, b1, W2, b2)
    return out[:, 0]


# speedup vs baseline: 1.2359x; 1.2359x over previous
import functools

import jax
import jax.numpy as jnp
from jax import lax
from jax.experimental import pallas as pl
from jax.experimental.pallas import tpu as pltpu
from jax.experimental.pallas import tpu_sc as plsc

B = 4096
L = 200
EMB = 64
HID = 128

NC = 2
NS = 16
NW = NC * NS
BPW = B // NW
C0 = 104
C1 = L - C0
NBUF = 4
UNROLL = 8

_mesh = plsc.VectorSubcoreMesh(core_axis_name="c", subcore_axis_name="s")


@functools.partial(
    pl.kernel,
    out_type=jax.ShapeDtypeStruct((B, EMB), jnp.float32),
    mesh=_mesh,
    compiler_params=pltpu.CompilerParams(use_tc_tiling_on_sc=False),
    scratch_types=[
        pltpu.VMEM((BPW * L,), jnp.int32),
        pltpu.VMEM((NBUF, L, EMB), jnp.float32),
        pltpu.VMEM((BPW, EMB), jnp.float32),
        pltpu.SemaphoreType.DMA,
        pltpu.SemaphoreType.DMA,
        pltpu.SemaphoreType.DMA,
        pltpu.SemaphoreType.DMA,
    ],
)
def _pool_sum(x_hbm, table_hbm, out_hbm, idx_v, rows_v, acc_v, s0, s1, s2, s3):
    sems = (s0, s1, s2, s3)
    wid = lax.axis_index("s") * NC + lax.axis_index("c")
    base = wid * (BPW * L)
    pltpu.sync_copy(x_hbm.at[pl.ds(base, BPW * L)], idx_v)

    def issue(i, b):
        off = i * L
        pltpu.async_copy(
            table_hbm.at[idx_v.at[pl.ds(off, C0)]], rows_v.at[b, pl.ds(0, C0)],
            sems[b],
        )
        pltpu.async_copy(
            table_hbm.at[idx_v.at[pl.ds(off + C0, C1)]],
            rows_v.at[b, pl.ds(C0, C1)],
            sems[b],
        )

    def wait(b):
        pltpu.make_async_copy(
            table_hbm.at[pl.ds(0, L)], rows_v.at[b], sems[b]
        ).wait()

    def reduce(i, b):
        def red(jo, accs):
            for u in range(UNROLL):
                j = jo * UNROLL + u
                accs = tuple(
                    accs[k] + rows_v[b, j, pl.ds(k * 16, 16)]
                    for k in range(EMB // 16)
                )
            return accs

        zero = jnp.zeros((16,), jnp.float32)
        accs = lax.fori_loop(0, L // UNROLL, red, (zero,) * (EMB // 16))
        for k in range(EMB // 16):
            acc_v[i, pl.ds(k * 16, 16)] = accs[k]

    for b in range(NBUF - 1):
        issue(b, b)

    def outer(g, _):
        i0 = g * NBUF
        for b in range(NBUF):
            i = i0 + b
            nxt = i + (NBUF - 1)

            @pl.when(nxt < BPW)
            def _():
                issue(nxt, (b + NBUF - 1) % NBUF)

            wait(b)
            reduce(i, b)
        return 0

    lax.fori_loop(0, BPW // NBUF, outer, 0)
    pltpu.sync_copy(acc_v, out_hbm.at[pl.ds(wid * BPW, BPW)])


def _mlp_body(pooled_ref, w1_ref, b1_ref, w2_ref, b2_ref, out_ref):
    s = jnp.dot(pooled_ref[...], w1_ref[...], preferred_element_type=jnp.float32)
    h = jnp.maximum(s * (1.0 / L) + b1_ref[...], 0.0)
    o = jnp.dot(h, w2_ref[...], preferred_element_type=jnp.float32)
    out_ref[...] = o + b2_ref[...]


_mlp = pl.pallas_call(
    _mlp_body,
    out_shape=jax.ShapeDtypeStruct((B, 1), jnp.float32),
)


def kernel(x, table, W1, b1, W2, b2):
    x_flat = x.reshape(-1).astype(jnp.int32)
    sums = _pool_sum(x_flat, table)
    out = _mlp(sums, W1, b1, W2, b2)
    return out[:, 0]


# speedup vs baseline: 1.2588x; 1.0185x over previous
import functools

import jax
import jax.numpy as jnp
from jax import lax
from jax.experimental import pallas as pl
from jax.experimental.pallas import tpu as pltpu
from jax.experimental.pallas import tpu_sc as plsc

B = 4096
L = 200
V = 1000000
EMB = 64
HID = 128

NC = 2
NS = 16
NW = NC * NS
BPW = B // NW
C0 = 104
C1 = L - C0
NBUF = 2
UNROLL = 4

GBLK = 2048


def _g_body(tt_ref, w1_ref, out_ref):
    a = tt_ref[...].astype(jnp.bfloat16)
    w = w1_ref[...].astype(jnp.bfloat16)
    out_ref[...] = jax.lax.dot_general(
        a, w, (((0,), (0,)), ((), ())), preferred_element_type=jnp.float32
    )


_g_call = pl.pallas_call(
    _g_body,
    grid=((V + GBLK - 1) // GBLK,),
    in_specs=[
        pl.BlockSpec((EMB, GBLK), lambda i: (0, i)),
        pl.BlockSpec((EMB, HID), lambda i: (0, 0)),
    ],
    out_specs=pl.BlockSpec((GBLK, HID), lambda i: (i, 0)),
    out_shape=jax.ShapeDtypeStruct((V, HID), jnp.float32),
)

_mesh = plsc.VectorSubcoreMesh(core_axis_name="c", subcore_axis_name="s")


@functools.partial(
    pl.kernel,
    out_type=jax.ShapeDtypeStruct((B, HID), jnp.float32),
    mesh=_mesh,
    scratch_types=[
        pltpu.VMEM((BPW * L,), jnp.int32),
        pltpu.VMEM((NBUF, L, HID), jnp.float32),
        pltpu.VMEM((BPW, HID), jnp.float32),
        pltpu.SemaphoreType.DMA,
        pltpu.SemaphoreType.DMA,
    ],
)
def _pool_sum(x_hbm, g_hbm, out_hbm, idx_v, rows_v, acc_v, s0, s1):
    sems = (s0, s1)
    wid = lax.axis_index("s") * NC + lax.axis_index("c")
    base = wid * (BPW * L)
    pltpu.sync_copy(x_hbm.at[pl.ds(base, BPW * L)], idx_v)

    def issue(i, b):
        off = i * L
        pltpu.async_copy(
            g_hbm.at[idx_v.at[pl.ds(off, C0)]], rows_v.at[b, pl.ds(0, C0)],
            sems[b],
        )
        pltpu.async_copy(
            g_hbm.at[idx_v.at[pl.ds(off + C0, C1)]],
            rows_v.at[b, pl.ds(C0, C1)],
            sems[b],
        )

    def wait(b):
        pltpu.make_async_copy(
            g_hbm.at[pl.ds(0, L)], rows_v.at[b], sems[b]
        ).wait()

    def reduce(i, b):
        def red(jo, accs):
            for u in range(UNROLL):
                j = jo * UNROLL + u
                accs = tuple(
                    accs[k] + rows_v[b, j, pl.ds(k * 16, 16)]
                    for k in range(HID // 16)
                )
            return accs

        zero = jnp.zeros((16,), jnp.float32)
        accs = lax.fori_loop(0, L // UNROLL, red, (zero,) * (HID // 16))
        for k in range(HID // 16):
            acc_v[i, pl.ds(k * 16, 16)] = accs[k]

    for b in range(NBUF - 1):
        issue(b, b)

    def outer(g, _):
        i0 = g * NBUF
        for b in range(NBUF):
            i = i0 + b
            nxt = i + (NBUF - 1)

            @pl.when(nxt < BPW)
            def _():
                issue(nxt, (b + NBUF - 1) % NBUF)

            wait(b)
            reduce(i, b)
        return 0

    lax.fori_loop(0, BPW // NBUF, outer, 0)
    pltpu.sync_copy(acc_v, out_hbm.at[pl.ds(wid * BPW, BPW)])


def _mlp_body(sums_ref, b1_ref, w2_ref, b2_ref, out_ref):
    h = jnp.maximum(sums_ref[...] * (1.0 / L) + b1_ref[...], 0.0)
    o = jnp.dot(h, w2_ref[...], preferred_element_type=jnp.float32)
    out_ref[...] = o + b2_ref[...]


_mlp = pl.pallas_call(
    _mlp_body,
    out_shape=jax.ShapeDtypeStruct((B, 1), jnp.float32),
)


def kernel(x, table, W1, b1, W2, b2):
    tt = table.T
    g = _g_call(tt, W1)
    x_flat = x.reshape(-1).astype(jnp.int32)
    sums = _pool_sum(x_flat, g)
    out = _mlp(sums, b1, W2, b2)
    return out[:, 0]


# speedup vs baseline: 1.8230x; 1.4483x over previous
import functools

import jax
import jax.numpy as jnp
from jax import lax
from jax.experimental import pallas as pl
from jax.experimental.pallas import tpu as pltpu
from jax.experimental.pallas import tpu_sc as plsc

B = 4096
L = 200
V = 1000000
EMB = 64
HID = 128

NC = 2
NS = 16
NW = NC * NS
BPW = B // NW
C0 = 104
C1 = L - C0
NBUF = 2
UNROLL = 8

GBLK = 8192


def _g_body(tt_ref, w1_ref, out_ref):
    a = tt_ref[...].astype(jnp.bfloat16)
    w = w1_ref[...].astype(jnp.bfloat16)
    out_ref[...] = jax.lax.dot_general(
        a, w, (((0,), (0,)), ((), ())), preferred_element_type=jnp.float32
    )


_g_call = pl.pallas_call(
    _g_body,
    grid=((V + GBLK - 1) // GBLK,),
    in_specs=[
        pl.BlockSpec((EMB, GBLK), lambda i: (0, i)),
        pl.BlockSpec((EMB, HID), lambda i: (0, 0)),
    ],
    out_specs=pl.BlockSpec((GBLK, HID), lambda i: (i, 0)),
    out_shape=jax.ShapeDtypeStruct((V, HID), jnp.float32),
)

_mesh = plsc.VectorSubcoreMesh(core_axis_name="c", subcore_axis_name="s")


@functools.partial(
    pl.kernel,
    out_type=jax.ShapeDtypeStruct((B, HID), jnp.float32),
    mesh=_mesh,
    scratch_types=[
        pltpu.VMEM((BPW * L,), jnp.int32),
        pltpu.VMEM((NBUF, L, HID), jnp.float32),
        pltpu.VMEM((BPW, HID), jnp.float32),
        pltpu.SemaphoreType.DMA,
        pltpu.SemaphoreType.DMA,
    ],
)
def _pool_sum(x_hbm, g_hbm, out_hbm, idx_v, rows_v, acc_v, s0, s1):
    sems = (s0, s1)
    wid = lax.axis_index("s") * NC + lax.axis_index("c")
    base = wid * (BPW * L)
    pltpu.sync_copy(x_hbm.at[pl.ds(base, BPW * L)], idx_v)

    def issue(i, b):
        off = i * L
        pltpu.async_copy(
            g_hbm.at[idx_v.at[pl.ds(off, C0)]], rows_v.at[b, pl.ds(0, C0)],
            sems[b],
        )
        pltpu.async_copy(
            g_hbm.at[idx_v.at[pl.ds(off + C0, C1)]],
            rows_v.at[b, pl.ds(C0, C1)],
            sems[b],
        )

    def wait(b):
        pltpu.make_async_copy(
            g_hbm.at[pl.ds(0, L)], rows_v.at[b], sems[b]
        ).wait()

    def reduce(i, b):
        def red(jo, accs):
            for u in range(UNROLL):
                j = jo * UNROLL + u
                accs = tuple(
                    accs[k] + rows_v[b, j, pl.ds(k * 16, 16)]
                    for k in range(HID // 16)
                )
            return accs

        zero = jnp.zeros((16,), jnp.float32)
        accs = lax.fori_loop(0, L // UNROLL, red, (zero,) * (HID // 16))
        for k in range(HID // 16):
            acc_v[i, pl.ds(k * 16, 16)] = accs[k]

    for b in range(NBUF - 1):
        issue(b, b)

    def outer(g, _):
        i0 = g * NBUF
        for b in range(NBUF):
            i = i0 + b
            nxt = i + (NBUF - 1)

            @pl.when(nxt < BPW)
            def _():
                issue(nxt, (b + NBUF - 1) % NBUF)

            wait(b)
            reduce(i, b)
        return 0

    lax.fori_loop(0, BPW // NBUF, outer, 0)
    pltpu.sync_copy(acc_v, out_hbm.at[pl.ds(wid * BPW, BPW)])


def _mlp_body(sums_ref, b1_ref, w2_ref, b2_ref, out_ref):
    h = jnp.maximum(sums_ref[...] * (1.0 / L) + b1_ref[...], 0.0)
    o = jnp.dot(h, w2_ref[...], preferred_element_type=jnp.float32)
    out_ref[...] = o + b2_ref[...]


_mlp = pl.pallas_call(
    _mlp_body,
    out_shape=jax.ShapeDtypeStruct((B, 1), jnp.float32),
)


def kernel(x, table, W1, b1, W2, b2):
    tt = table.T
    g = _g_call(tt, W1)
    x_flat = x.reshape(-1).astype(jnp.int32)
    sums = _pool_sum(x_flat, g)
    out = _mlp(sums, b1, W2, b2)
    return out[:, 0]


# speedup vs baseline: 2.0413x; 1.1197x over previous
import functools

import jax
import jax.numpy as jnp
from jax import lax
from jax.experimental import pallas as pl
from jax.experimental.pallas import tpu as pltpu
from jax.experimental.pallas import tpu_sc as plsc

B = 4096
L = 200
V = 1000000
EMB = 64
HID = 128

NC = 2
NS = 16
NW = NC * NS
BPW = B // NW
C0 = 104
C1 = L - C0
NSLOT = 4
NUNIT = 2 * BPW

GBLK = 16384


def _g_body(tt_ref, w1_ref, out_ref):
    a = tt_ref[...].astype(jnp.bfloat16)
    w = w1_ref[...].astype(jnp.bfloat16)
    out_ref[...] = jax.lax.dot_general(
        a, w, (((0,), (0,)), ((), ())), preferred_element_type=jnp.float32
    )


_g_call = pl.pallas_call(
    _g_body,
    grid=((V + GBLK - 1) // GBLK,),
    in_specs=[
        pl.BlockSpec((EMB, GBLK), lambda i: (0, i)),
        pl.BlockSpec((EMB, HID), lambda i: (0, 0)),
    ],
    out_specs=pl.BlockSpec((GBLK, HID), lambda i: (i, 0)),
    out_shape=jax.ShapeDtypeStruct((V, HID), jnp.float32),
)

_mesh = plsc.VectorSubcoreMesh(core_axis_name="c", subcore_axis_name="s")


@functools.partial(
    pl.kernel,
    out_type=jax.ShapeDtypeStruct((B, HID), jnp.float32),
    mesh=_mesh,
    scratch_types=[
        pltpu.VMEM((BPW * L,), jnp.int32),
        pltpu.VMEM((NSLOT, C0, HID), jnp.float32),
        pltpu.VMEM((BPW, HID), jnp.float32),
        pltpu.SemaphoreType.DMA,
        pltpu.SemaphoreType.DMA,
        pltpu.SemaphoreType.DMA,
        pltpu.SemaphoreType.DMA,
    ],
)
def _pool_sum(x_hbm, g_hbm, out_hbm, idx_v, rows_v, acc_v, s0, s1, s2, s3):
    sems = (s0, s1, s2, s3)
    wid = lax.axis_index("s") * NC + lax.axis_index("c")
    base = wid * (BPW * L)
    pltpu.sync_copy(x_hbm.at[pl.ds(base, BPW * L)], idx_v)

    def unit_len(b):
        return C0 if b % 2 == 0 else C1

    def issue(u, b):
        ln = unit_len(b)
        off = (u >> 1) * L + (0 if b % 2 == 0 else C0)
        pltpu.async_copy(
            g_hbm.at[idx_v.at[pl.ds(off, ln)]],
            rows_v.at[b, pl.ds(0, ln)],
            sems[b],
        )

    def wait(b):
        ln = unit_len(b)
        pltpu.make_async_copy(
            g_hbm.at[pl.ds(0, ln)], rows_v.at[b, pl.ds(0, ln)], sems[b]
        ).wait()

    def reduce_unit(b, accs):
        ln = unit_len(b)

        def red(jo, accs):
            for u8 in range(8):
                j = jo * 8 + u8
                accs = tuple(
                    accs[k] + rows_v[b, j, pl.ds(k * 16, 16)]
                    for k in range(HID // 16)
                )
            return accs

        return lax.fori_loop(0, ln // 8, red, accs)

    for b in range(NSLOT - 1):
        issue(b, b)

    zeros = (jnp.zeros((16,), jnp.float32),) * (HID // 16)

    def outer(g, _):
        u0 = g * NSLOT
        accs = zeros
        for b in range(NSLOT):
            u = u0 + b
            nxt = u + (NSLOT - 1)

            @pl.when(nxt < NUNIT)
            def _():
                issue(nxt, (b + NSLOT - 1) % NSLOT)

            wait(b)
            if b % 2 == 0:
                accs = reduce_unit(b, zeros)
            else:
                accs = reduce_unit(b, accs)
                row = u >> 1
                for k in range(HID // 16):
                    acc_v[row, pl.ds(k * 16, 16)] = accs[k]
        return 0

    lax.fori_loop(0, NUNIT // NSLOT, outer, 0)
    pltpu.sync_copy(acc_v, out_hbm.at[pl.ds(wid * BPW, BPW)])


def _mlp_body(sums_ref, b1_ref, w2_ref, b2_ref, out_ref):
    h = jnp.maximum(sums_ref[...] * (1.0 / L) + b1_ref[...], 0.0)
    o = jnp.dot(h, w2_ref[...], preferred_element_type=jnp.float32)
    out_ref[...] = o + b2_ref[...]


_mlp = pl.pallas_call(
    _mlp_body,
    out_shape=jax.ShapeDtypeStruct((B, 1), jnp.float32),
)


def kernel(x, table, W1, b1, W2, b2):
    tt = table.T
    g = _g_call(tt, W1)
    x_flat = x.reshape(-1).astype(jnp.int32)
    sums = _pool_sum(x_flat, g)
    out = _mlp(sums, b1, W2, b2)
    return out[:, 0]


# speedup vs baseline: 2.0515x; 1.0050x over previous
import functools

import jax
import jax.numpy as jnp
from jax import lax
from jax.experimental import pallas as pl
from jax.experimental.pallas import tpu as pltpu
from jax.experimental.pallas import tpu_sc as plsc

B = 4096
L = 200
V = 1000000
EMB = 64
HID = 128

NC = 2
NS = 16
NW = NC * NS
BPW = B // NW
ULEN = (56, 48, 48, 48)
UOFF = (0, 56, 104, 152)
NSLOT = 8
UPR = 4
NUNIT = UPR * BPW

GBLK = 16384


def _g_body(tt_ref, w1_ref, out_ref):
    a = tt_ref[...].astype(jnp.bfloat16)
    w = w1_ref[...].astype(jnp.bfloat16)
    out_ref[...] = jax.lax.dot_general(
        a, w, (((0,), (0,)), ((), ())), preferred_element_type=jnp.float32
    )


_g_call = pl.pallas_call(
    _g_body,
    grid=((V + GBLK - 1) // GBLK,),
    in_specs=[
        pl.BlockSpec((EMB, GBLK), lambda i: (0, i)),
        pl.BlockSpec((EMB, HID), lambda i: (0, 0)),
    ],
    out_specs=pl.BlockSpec((GBLK, HID), lambda i: (i, 0)),
    out_shape=jax.ShapeDtypeStruct((V, HID), jnp.float32),
)

_mesh = plsc.VectorSubcoreMesh(core_axis_name="c", subcore_axis_name="s")


@functools.partial(
    pl.kernel,
    out_type=jax.ShapeDtypeStruct((B, HID), jnp.float32),
    mesh=_mesh,
    scratch_types=[
        pltpu.VMEM((BPW * L,), jnp.int32),
        pltpu.VMEM((NSLOT, ULEN[0], HID), jnp.float32),
        pltpu.VMEM((BPW, HID), jnp.float32),
    ] + [pltpu.SemaphoreType.DMA] * NSLOT,
)
def _pool_sum(x_hbm, g_hbm, out_hbm, idx_v, rows_v, acc_v, *sems):
    wid = lax.axis_index("s") * NC + lax.axis_index("c")
    base = wid * (BPW * L)
    pltpu.sync_copy(x_hbm.at[pl.ds(base, BPW * L)], idx_v)

    def issue(u, b):
        q = b % UPR
        ln = ULEN[q]
        off = (u >> 2) * L + UOFF[q]
        pltpu.async_copy(
            g_hbm.at[idx_v.at[pl.ds(off, ln)]],
            rows_v.at[b, pl.ds(0, ln)],
            sems[b],
        )

    def wait(b):
        ln = ULEN[b % UPR]
        pltpu.make_async_copy(
            g_hbm.at[pl.ds(0, ln)], rows_v.at[b, pl.ds(0, ln)], sems[b]
        ).wait()

    def reduce_unit(b, accs):
        ln = ULEN[b % UPR]

        def red(jo, accs):
            for u8 in range(8):
                j = jo * 8 + u8
                accs = tuple(
                    accs[k] + rows_v[b, j, pl.ds(k * 16, 16)]
                    for k in range(HID // 16)
                )
            return accs

        return lax.fori_loop(0, ln // 8, red, accs)

    for b in range(NSLOT - 1):
        issue(b, b)

    zeros = (jnp.zeros((16,), jnp.float32),) * (HID // 16)

    def outer(g, _):
        u0 = g * NSLOT
        accs = zeros
        for b in range(NSLOT):
            u = u0 + b
            nxt = u + (NSLOT - 1)

            @pl.when(nxt < NUNIT)
            def _():
                issue(nxt, (b + NSLOT - 1) % NSLOT)

            wait(b)
            if b % UPR == 0:
                accs = reduce_unit(b, zeros)
            else:
                accs = reduce_unit(b, accs)
            if b % UPR == UPR - 1:
                row = u >> 2
                for k in range(HID // 16):
                    acc_v[row, pl.ds(k * 16, 16)] = accs[k]
        return 0

    lax.fori_loop(0, NUNIT // NSLOT, outer, 0)
    pltpu.sync_copy(acc_v, out_hbm.at[pl.ds(wid * BPW, BPW)])


def _mlp_body(sums_ref, b1_ref, w2_ref, b2_ref, out_ref):
    h = jnp.maximum(sums_ref[...] * (1.0 / L) + b1_ref[...], 0.0)
    o = jnp.dot(h, w2_ref[...], preferred_element_type=jnp.float32)
    out_ref[...] = o + b2_ref[...]


_mlp = pl.pallas_call(
    _mlp_body,
    out_shape=jax.ShapeDtypeStruct((B, 1), jnp.float32),
)


def kernel(x, table, W1, b1, W2, b2):
    tt = table.T
    g = _g_call(tt, W1)
    x_flat = x.reshape(-1).astype(jnp.int32)
    sums = _pool_sum(x_flat, g)
    out = _mlp(sums, b1, W2, b2)
    return out[:, 0]


# speedup vs baseline: 2.1055x; 1.0263x over previous
import functools

import jax
import jax.numpy as jnp
from jax import lax
from jax.experimental import pallas as pl
from jax.experimental.pallas import tpu as pltpu
from jax.experimental.pallas import tpu_sc as plsc

B = 4096
L = 200
V = 1000000
EMB = 64
HID = 128

NC = 2
NS = 16
NW = NC * NS
BPW = B // NW
ULEN = (56, 48, 48, 48)
UOFF = (0, 56, 104, 152)
NSLOT = 8
UPR = 4
NUNIT = UPR * BPW

GBLK = 32768


def _g_body(tt_ref, w1_ref, out_ref):
    a = tt_ref[...].astype(jnp.bfloat16)
    w = w1_ref[...].astype(jnp.bfloat16)
    out_ref[...] = jax.lax.dot_general(
        a, w, (((0,), (0,)), ((), ())), preferred_element_type=jnp.float32
    )


_g_call = pl.pallas_call(
    _g_body,
    grid=((V + GBLK - 1) // GBLK,),
    in_specs=[
        pl.BlockSpec((EMB, GBLK), lambda i: (0, i)),
        pl.BlockSpec((EMB, HID), lambda i: (0, 0)),
    ],
    out_specs=pl.BlockSpec((GBLK, HID), lambda i: (i, 0)),
    out_shape=jax.ShapeDtypeStruct((V, HID), jnp.float32),
)

_mesh = plsc.VectorSubcoreMesh(core_axis_name="c", subcore_axis_name="s")


@functools.partial(
    pl.kernel,
    out_type=jax.ShapeDtypeStruct((B, HID), jnp.float32),
    mesh=_mesh,
    scratch_types=[
        pltpu.VMEM((BPW * L,), jnp.int32),
        pltpu.VMEM((NSLOT, ULEN[0], HID), jnp.float32),
        pltpu.VMEM((BPW, HID), jnp.float32),
    ] + [pltpu.SemaphoreType.DMA] * NSLOT,
)
def _pool_sum(x_hbm, g_hbm, out_hbm, idx_v, rows_v, acc_v, *sems):
    wid = lax.axis_index("s") * NC + lax.axis_index("c")
    base = wid * (BPW * L)
    pltpu.sync_copy(x_hbm.at[pl.ds(base, BPW * L)], idx_v)

    def issue(u, b):
        q = b % UPR
        ln = ULEN[q]
        off = (u >> 2) * L + UOFF[q]
        pltpu.async_copy(
            g_hbm.at[idx_v.at[pl.ds(off, ln)]],
            rows_v.at[b, pl.ds(0, ln)],
            sems[b],
        )

    def wait(b):
        ln = ULEN[b % UPR]
        pltpu.make_async_copy(
            g_hbm.at[pl.ds(0, ln)], rows_v.at[b, pl.ds(0, ln)], sems[b]
        ).wait()

    def reduce_unit(b, accs):
        ln = ULEN[b % UPR]

        def red(jo, accs):
            for u8 in range(8):
                j = jo * 8 + u8
                accs = tuple(
                    accs[k] + rows_v[b, j, pl.ds(k * 16, 16)]
                    for k in range(HID // 16)
                )
            return accs

        return lax.fori_loop(0, ln // 8, red, accs)

    for b in range(NSLOT - 1):
        issue(b, b)

    zeros = (jnp.zeros((16,), jnp.float32),) * (HID // 16)

    def outer(g, _):
        u0 = g * NSLOT
        accs = zeros
        for b in range(NSLOT):
            u = u0 + b
            nxt = u + (NSLOT - 1)

            @pl.when(nxt < NUNIT)
            def _():
                issue(nxt, (b + NSLOT - 1) % NSLOT)

            wait(b)
            if b % UPR == 0:
                accs = reduce_unit(b, zeros)
            else:
                accs = reduce_unit(b, accs)
            if b % UPR == UPR - 1:
                row = u >> 2
                for k in range(HID // 16):
                    acc_v[row, pl.ds(k * 16, 16)] = accs[k]
        return 0

    lax.fori_loop(0, NUNIT // NSLOT, outer, 0)
    pltpu.sync_copy(acc_v, out_hbm.at[pl.ds(wid * BPW, BPW)])


def _mlp_body(sums_ref, b1_ref, w2_ref, b2_ref, out_ref):
    h = jnp.maximum(sums_ref[...] * (1.0 / L) + b1_ref[...], 0.0)
    o = jnp.sum(h * w2_ref[...][:, 0], axis=1) + b2_ref[0]
    out_ref[...] = o


_mlp = pl.pallas_call(
    _mlp_body,
    out_shape=jax.ShapeDtypeStruct((B,), jnp.float32),
)


def kernel(x, table, W1, b1, W2, b2):
    tt = table.T
    g = _g_call(tt, W1)
    x_flat = x.reshape(-1).astype(jnp.int32)
    sums = _pool_sum(x_flat, g)
    return _mlp(sums, b1, W2, b2)
